# baseline (speedup 1.0000x reference)
import functools

import jax
import jax.numpy as jnp
from jax import lax
from jax.experimental import pallas as pl
from jax.experimental.pallas import tpu as pltpu
from jax.experimental.pallas import tpu_sc as plsc

NB, BSZ, NG, BPG = 8, 4, 4, 2
L = 16
G = 4
NC, NS = 2, 16
NPS = 628
CH_MAX = NPS // G
CH_PAD = 160


def _sigmoid(x):
    return 1.0 / (1.0 + jnp.exp(-x))


def _tanh(x):
    return 1.0 - 2.0 / (jnp.exp(2.0 * x) + 1.0)


def _step_kernel_body(N, C, D, K,
                      msg_src, h_in, key_f, prim_f, dec_f, idx_f, bw_f, gw_f, cc_f,
                      h_out, msg_out,
                      idx_all, msgs_v, bw_v, gw_v, key_v, prim_v, h_v, dec_v,
                      cc_v, sim_b, hn_v, mn_v, sem_g, sem_l, sem_o, sem_i):
    nch = D // L
    bs = lax.axis_index("c")
    sid = lax.axis_index("s")
    n_start = sid * NPS
    nodes_here = jnp.minimum(NPS, N - n_start)
    ch_count = nodes_here // G

    row0 = (bs * NS + sid) * CH_PAD
    pltpu.async_copy(idx_f.at[pl.ds(row0, CH_PAD)], idx_all, sem_i)
    pltpu.async_copy(cc_f.at[pl.ds(bs * C, C)], cc_v, sem_i)
    pltpu.make_async_copy(idx_f.at[pl.ds(0, CH_PAD)], idx_all, sem_i).wait()
    pltpu.make_async_copy(cc_f.at[pl.ds(0, C)], cc_v, sem_i).wait()

    idx3 = idx_all

    def issue(j, p):
        @pl.when(j < ch_count)
        def _():
            n0 = n_start + j * G
            base = bs * N + n0
            pltpu.async_copy(msg_src.at[idx3.at[j]], msgs_v.at[p], sem_g.at[p])
            pltpu.async_copy(bw_f.at[pl.ds(n0, G)], bw_v.at[p], sem_l.at[p])
            pltpu.async_copy(gw_f.at[pl.ds(n0, G)], gw_v.at[p], sem_l.at[p])
            pltpu.async_copy(key_f.at[pl.ds(base, G)], key_v.at[p], sem_l.at[p])
            pltpu.async_copy(prim_f.at[pl.ds(base, G)], prim_v.at[p], sem_l.at[p])
            pltpu.async_copy(h_in.at[pl.ds(base, G)], h_v.at[p], sem_l.at[p])
            pltpu.async_copy(dec_f.at[pl.ds(base, G)], dec_v.at[p], sem_l.at[p])

    def wait_in(j, p):
        pltpu.make_async_copy(msg_src.at[idx3.at[j]], msgs_v.at[p],
                              sem_g.at[p]).wait()
        n0 = n_start + j * G
        base = bs * N + n0
        pltpu.make_async_copy(bw_f.at[pl.ds(n0, G)], bw_v.at[p], sem_l.at[p]).wait()
        pltpu.make_async_copy(gw_f.at[pl.ds(n0, G)], gw_v.at[p], sem_l.at[p]).wait()
        pltpu.make_async_copy(key_f.at[pl.ds(base, G)], key_v.at[p], sem_l.at[p]).wait()
        pltpu.make_async_copy(prim_f.at[pl.ds(base, G)], prim_v.at[p], sem_l.at[p]).wait()
        pltpu.make_async_copy(h_in.at[pl.ds(base, G)], h_v.at[p], sem_l.at[p]).wait()
        pltpu.make_async_copy(dec_f.at[pl.ds(base, G)], dec_v.at[p], sem_l.at[p]).wait()

    def wait_out(j, p):
        n0 = n_start + j * G
        base = bs * N + n0
        pltpu.make_async_copy(hn_v.at[p], h_out.at[pl.ds(base, G)],
                              sem_o.at[p]).wait()
        pltpu.make_async_copy(mn_v.at[p], msg_out.at[pl.ds(base, G)],
                              sem_o.at[p]).wait()

    issue(0, 0)

    def chunk_iter(j, _):
        p = j % 2
        issue(j + 1, (j + 1) % 2)

        @pl.when(j < ch_count)
        def _():
            wait_in(j, p)
            @pl.when(j >= 2)
            def _():
                wait_out(j - 2, p)

            @plsc.parallel_loop(0, G, unroll=2)
            def node_iter(i):
                node = n_start + j * G + i
                kv = [key_v[p, i, pl.ds(c * L, L)] for c in range(nch)]

                recv = [jnp.zeros((L,), jnp.float32) for _ in range(nch)]
                for g in range(NG):
                    gacc = [jnp.zeros((L,), jnp.float32) for _ in range(nch)]
                    for bp in range(BPG):
                        b = g * BPG + bp
                        bacc = [jnp.zeros((L,), jnp.float32) for _ in range(nch)]
                        for s in range(BSZ):
                            kk = b * BSZ + s
                            row = i * K + kk
                            m = [msgs_v[p, row, pl.ds(c * L, L)]
                                 for c in range(nch)]
                            e = [m[c] * kv[c] for c in range(nch)]
                            part = (((e[0] + e[1]) + (e[2] + e[3]))
                                    + ((e[4] + e[5]) + (e[6] + e[7])))
                            sim = jnp.sum(part)
                            rt = _sigmoid(jnp.broadcast_to(sim, (L,)))
                            for c in range(nch):
                                w = bw_v[p, i, kk, pl.ds(c * L, L)]
                                bacc[c] = bacc[c] + rt * (m[c] * w)
                        for c in range(nch):
                            gw = gw_v[p, i, b, pl.ds(c * L, L)]
                            gacc[c] = gacc[c] + _tanh(bacc[c]) * gw
                    for c in range(nch):
                        recv[c] = recv[c] + _tanh(gacc[c])

                inv_ng = 1.0 / NG

                def add_cc(rv):
                    ccn = jnp.minimum(node, C - 1)
                    return [rv[c] + cc_v[ccn, pl.ds(c * L, L)]
                            for c in range(nch)]

                recv = lax.cond(node < C, add_cc, lambda rv: list(rv),
                                [r * inv_ng for r in recv])

                for c in range(nch):
                    sl = pl.ds(c * L, L)
                    dv = dec_v[p, i]
                    hn = dv * h_v[p, i, sl] + (1.0 - dv) * recv[c]
                    hn_v[p, i, sl] = hn
                    mn_v[p, i, sl] = _tanh(hn * prim_v[p, i, sl])

            n0 = n_start + j * G
            base = bs * N + n0
            pltpu.async_copy(hn_v.at[p], h_out.at[pl.ds(base, G)], sem_o.at[p])
            pltpu.async_copy(mn_v.at[p], msg_out.at[pl.ds(base, G)], sem_o.at[p])

        return 0

    lax.fori_loop(0, CH_MAX, chunk_iter, 0)

    @pl.when(ch_count >= 2)
    def _():
        wait_out(ch_count - 2, (ch_count - 2) % 2)

    @pl.when(ch_count >= 1)
    def _():
        wait_out(ch_count - 1, (ch_count - 1) % 2)


@functools.partial(jax.jit, static_argnums=(9, 10, 11, 12))
def _step(msg_f, h_f, key_f, prim_f, dec_f, idx_f, bw_f, gw_f, cc_f, N, C, D, K):
    BSN = msg_f.shape[0]
    mesh = plsc.VectorSubcoreMesh(core_axis_name="c", subcore_axis_name="s",
                                  num_cores=NC, num_subcores=NS)
    body = functools.partial(_step_kernel_body, N, C, D, K)
    return pl.kernel(
        body,
        out_type=(
            jax.ShapeDtypeStruct((BSN, D), jnp.float32),
            jax.ShapeDtypeStruct((BSN, D), jnp.float32),
        ),
        mesh=mesh,
        compiler_params=pltpu.CompilerParams(needs_layout_passes=False),
        scratch_types=[
            pltpu.VMEM((CH_PAD, G * K), jnp.int32),
            pltpu.VMEM((2, G * K, D), jnp.float32),
            pltpu.VMEM((2, G, NB * BSZ, D), jnp.float32),
            pltpu.VMEM((2, G, NG * BPG, D), jnp.float32),
            pltpu.VMEM((2, G, D), jnp.float32),
            pltpu.VMEM((2, G, D), jnp.float32),
            pltpu.VMEM((2, G, D), jnp.float32),
            pltpu.VMEM((2, G, L), jnp.float32),
            pltpu.VMEM((16, D), jnp.float32),
            pltpu.VMEM((L, L), jnp.float32),
            pltpu.VMEM((2, G, D), jnp.float32),
            pltpu.VMEM((2, G, D), jnp.float32),
            pltpu.SemaphoreType.DMA((2,)),
            pltpu.SemaphoreType.DMA((2,)),
            pltpu.SemaphoreType.DMA((2,)),
            pltpu.SemaphoreType.DMA,
        ],
    )(msg_f, h_f, key_f, prim_f, dec_f, idx_f, bw_f, gw_f, cc_f)


def kernel(cc_signals, h_prev, prev_messages, eff_prim, eff_key, eff_decay,
           conn_indices, branch_w, group_w):
    BS, T, C, D = cc_signals.shape
    N, K = conn_indices.shape
    n_pad = NS * NPS

    conn = conn_indices.astype(jnp.int32)
    conn = jnp.pad(conn, ((0, n_pad - N), (0, 0)))
    idx_f = (conn[None] + (jnp.arange(BS, dtype=jnp.int32) * N)[:, None, None])
    idx_f = idx_f.reshape(BS, NS, CH_MAX, G * K)
    idx_f = jnp.pad(idx_f, ((0, 0), (0, 0), (0, CH_PAD - CH_MAX), (0, 0)))
    idx_f = idx_f.reshape(BS * NS * CH_PAD, G * K)
    dec_f = jnp.broadcast_to(eff_decay[..., None], (BS, N, L)).reshape(BS * N, L)
    h_f = h_prev.reshape(BS * N, D)
    msg_f = prev_messages.reshape(BS * N, D)
    key_f = eff_key.reshape(BS * N, D)
    prim_f = eff_prim.reshape(BS * N, D)
    bw_f = branch_w.reshape(N, NB * BSZ, D)
    gw_f = group_w.reshape(N, NG * BPG, D)

    outs = []
    h, m = h_f, msg_f
    for t in range(T):
        cc_f = cc_signals[:, t].reshape(BS * C, D)
        h, m = _step(m, h, key_f, prim_f, dec_f, idx_f, bw_f, gw_f, cc_f,
                     N, C, D, K)
        outs.append(m.reshape(BS, N, D)[:, :C])

    output = jnp.stack(outs, axis=1)
    return output, h.reshape(BS, N, D)


# speedup vs baseline: 1.1091x; 1.1091x over previous
import functools

import jax
import jax.numpy as jnp
from jax import lax
from jax.experimental import pallas as pl
from jax.experimental.pallas import tpu as pltpu
from jax.experimental.pallas import tpu_sc as plsc

NB, BSZ, NG, BPG = 8, 4, 4, 2
L = 16
G = 4
NC, NS = 2, 16
NPS = 628
CH_MAX = NPS // G
CH_PAD = 160


def _sigmoid(x):
    return 1.0 / (1.0 + jnp.exp(-x))


def _tanh(x):
    return 1.0 - 2.0 / (jnp.exp(2.0 * x) + 1.0)


def _step_kernel_body(N, C, D, K,
                      msg_src, h_in, key_f, prim_f, dec_f, idx_f, bw_f, gw_f, cc_f,
                      h_out, msg_out,
                      idx_all, msgs_v, bw_v, gw_v, key_v, prim_v, h_v, dec_v,
                      cc_v, sim_b, hn_v, mn_v, sem_g, sem_l, sem_o, sem_i):
    nch = D // L
    bs = lax.axis_index("c")
    sid = lax.axis_index("s")
    n_start = sid * NPS
    nodes_here = jnp.minimum(NPS, N - n_start)
    ch_count = nodes_here // G

    row0 = (bs * NS + sid) * CH_PAD
    pltpu.async_copy(idx_f.at[pl.ds(row0, CH_PAD)], idx_all, sem_i)
    pltpu.async_copy(cc_f.at[pl.ds(bs * C, C)], cc_v, sem_i)
    pltpu.make_async_copy(idx_f.at[pl.ds(0, CH_PAD)], idx_all, sem_i).wait()
    pltpu.make_async_copy(cc_f.at[pl.ds(0, C)], cc_v, sem_i).wait()

    idx3 = idx_all

    def issue(j, p):
        @pl.when(j < ch_count)
        def _():
            n0 = n_start + j * G
            base = bs * N + n0
            pltpu.async_copy(msg_src.at[idx3.at[j]], msgs_v.at[p], sem_g.at[p])
            pltpu.async_copy(bw_f.at[pl.ds(n0, G)], bw_v.at[p], sem_l.at[p])
            pltpu.async_copy(gw_f.at[pl.ds(n0, G)], gw_v.at[p], sem_l.at[p])
            pltpu.async_copy(key_f.at[pl.ds(base, G)], key_v.at[p], sem_l.at[p])
            pltpu.async_copy(prim_f.at[pl.ds(base, G)], prim_v.at[p], sem_l.at[p])
            pltpu.async_copy(h_in.at[pl.ds(base, G)], h_v.at[p], sem_l.at[p])
            pltpu.async_copy(dec_f.at[pl.ds(base, G)], dec_v.at[p], sem_l.at[p])

    def wait_in(j, p):
        pltpu.make_async_copy(msg_src.at[idx3.at[j]], msgs_v.at[p],
                              sem_g.at[p]).wait()
        n0 = n_start + j * G
        base = bs * N + n0
        pltpu.make_async_copy(bw_f.at[pl.ds(n0, G)], bw_v.at[p], sem_l.at[p]).wait()
        pltpu.make_async_copy(gw_f.at[pl.ds(n0, G)], gw_v.at[p], sem_l.at[p]).wait()
        pltpu.make_async_copy(key_f.at[pl.ds(base, G)], key_v.at[p], sem_l.at[p]).wait()
        pltpu.make_async_copy(prim_f.at[pl.ds(base, G)], prim_v.at[p], sem_l.at[p]).wait()
        pltpu.make_async_copy(h_in.at[pl.ds(base, G)], h_v.at[p], sem_l.at[p]).wait()
        pltpu.make_async_copy(dec_f.at[pl.ds(base, G)], dec_v.at[p], sem_l.at[p]).wait()

    def wait_out(j, p):
        n0 = n_start + j * G
        base = bs * N + n0
        pltpu.make_async_copy(hn_v.at[p], h_out.at[pl.ds(base, G)],
                              sem_o.at[p]).wait()
        pltpu.make_async_copy(mn_v.at[p], msg_out.at[pl.ds(base, G)],
                              sem_o.at[p]).wait()

    issue(0, 0)

    def chunk_iter(j, _):
        p = j % 2
        issue(j + 1, (j + 1) % 2)

        @pl.when(j < ch_count)
        def _():
            wait_in(j, p)
            @pl.when(j >= 2)
            def _():
                wait_out(j - 2, p)

            @plsc.parallel_loop(0, G, unroll=1)
            def node_iter(i):
                node = n_start + j * G + i
                kv = [key_v[p, i, pl.ds(c * L, L)] for c in range(nch)]

                recv = [jnp.zeros((L,), jnp.float32) for _ in range(nch)]
                for g in range(NG):
                    gacc = [jnp.zeros((L,), jnp.float32) for _ in range(nch)]
                    for bp in range(BPG):
                        b = g * BPG + bp
                        ms = []
                        for s in range(BSZ):
                            row = i * K + b * BSZ + s
                            ms.append([msgs_v[p, row, pl.ds(c * L, L)]
                                       for c in range(nch)])
                        parts = []
                        for s in range(BSZ):
                            e = [ms[s][c] * kv[c] for c in range(nch)]
                            parts.append((((e[0] + e[1]) + (e[2] + e[3]))
                                          + ((e[4] + e[5]) + (e[6] + e[7]))))
                        sims = [jnp.sum(parts[s]) for s in range(BSZ)]
                        rts = [_sigmoid(jnp.broadcast_to(sims[s], (L,)))
                               for s in range(BSZ)]
                        bacc = [jnp.zeros((L,), jnp.float32) for _ in range(nch)]
                        for s in range(BSZ):
                            kk = b * BSZ + s
                            for c in range(nch):
                                w = bw_v[p, i, kk, pl.ds(c * L, L)]
                                bacc[c] = bacc[c] + rts[s] * (ms[s][c] * w)
                        for c in range(nch):
                            gw = gw_v[p, i, b, pl.ds(c * L, L)]
                            gacc[c] = gacc[c] + _tanh(bacc[c]) * gw
                    for c in range(nch):
                        recv[c] = recv[c] + _tanh(gacc[c])

                inv_ng = 1.0 / NG

                def add_cc(rv):
                    ccn = jnp.minimum(node, C - 1)
                    return [rv[c] + cc_v[ccn, pl.ds(c * L, L)]
                            for c in range(nch)]

                recv = lax.cond(node < C, add_cc, lambda rv: list(rv),
                                [r * inv_ng for r in recv])

                for c in range(nch):
                    sl = pl.ds(c * L, L)
                    dv = dec_v[p, i]
                    hn = dv * h_v[p, i, sl] + (1.0 - dv) * recv[c]
                    hn_v[p, i, sl] = hn
                    mn_v[p, i, sl] = _tanh(hn * prim_v[p, i, sl])

            n0 = n_start + j * G
            base = bs * N + n0
            pltpu.async_copy(hn_v.at[p], h_out.at[pl.ds(base, G)], sem_o.at[p])
            pltpu.async_copy(mn_v.at[p], msg_out.at[pl.ds(base, G)], sem_o.at[p])

        return 0

    lax.fori_loop(0, CH_MAX, chunk_iter, 0)

    @pl.when(ch_count >= 2)
    def _():
        wait_out(ch_count - 2, (ch_count - 2) % 2)

    @pl.when(ch_count >= 1)
    def _():
        wait_out(ch_count - 1, (ch_count - 1) % 2)


@functools.partial(jax.jit, static_argnums=(9, 10, 11, 12))
def _step(msg_f, h_f, key_f, prim_f, dec_f, idx_f, bw_f, gw_f, cc_f, N, C, D, K):
    BSN = msg_f.shape[0]
    mesh = plsc.VectorSubcoreMesh(core_axis_name="c", subcore_axis_name="s",
                                  num_cores=NC, num_subcores=NS)
    body = functools.partial(_step_kernel_body, N, C, D, K)
    return pl.kernel(
        body,
        out_type=(
            jax.ShapeDtypeStruct((BSN, D), jnp.float32),
            jax.ShapeDtypeStruct((BSN, D), jnp.float32),
        ),
        mesh=mesh,
        compiler_params=pltpu.CompilerParams(needs_layout_passes=False),
        scratch_types=[
            pltpu.VMEM((CH_PAD, G * K), jnp.int32),
            pltpu.VMEM((2, G * K, D), jnp.float32),
            pltpu.VMEM((2, G, NB * BSZ, D), jnp.float32),
            pltpu.VMEM((2, G, NG * BPG, D), jnp.float32),
            pltpu.VMEM((2, G, D), jnp.float32),
            pltpu.VMEM((2, G, D), jnp.float32),
            pltpu.VMEM((2, G, D), jnp.float32),
            pltpu.VMEM((2, G, L), jnp.float32),
            pltpu.VMEM((16, D), jnp.float32),
            pltpu.VMEM((L, L), jnp.float32),
            pltpu.VMEM((2, G, D), jnp.float32),
            pltpu.VMEM((2, G, D), jnp.float32),
            pltpu.SemaphoreType.DMA((2,)),
            pltpu.SemaphoreType.DMA((2,)),
            pltpu.SemaphoreType.DMA((2,)),
            pltpu.SemaphoreType.DMA,
        ],
    )(msg_f, h_f, key_f, prim_f, dec_f, idx_f, bw_f, gw_f, cc_f)


def kernel(cc_signals, h_prev, prev_messages, eff_prim, eff_key, eff_decay,
           conn_indices, branch_w, group_w):
    BS, T, C, D = cc_signals.shape
    N, K = conn_indices.shape
    n_pad = NS * NPS

    conn = conn_indices.astype(jnp.int32)
    conn = jnp.pad(conn, ((0, n_pad - N), (0, 0)))
    idx_f = (conn[None] + (jnp.arange(BS, dtype=jnp.int32) * N)[:, None, None])
    idx_f = idx_f.reshape(BS, NS, CH_MAX, G * K)
    idx_f = jnp.pad(idx_f, ((0, 0), (0, 0), (0, CH_PAD - CH_MAX), (0, 0)))
    idx_f = idx_f.reshape(BS * NS * CH_PAD, G * K)
    dec_f = jnp.broadcast_to(eff_decay[..., None], (BS, N, L)).reshape(BS * N, L)
    h_f = h_prev.reshape(BS * N, D)
    msg_f = prev_messages.reshape(BS * N, D)
    key_f = eff_key.reshape(BS * N, D)
    prim_f = eff_prim.reshape(BS * N, D)
    bw_f = branch_w.reshape(N, NB * BSZ, D)
    gw_f = group_w.reshape(N, NG * BPG, D)

    outs = []
    h, m = h_f, msg_f
    for t in range(T):
        cc_f = cc_signals[:, t].reshape(BS * C, D)
        h, m = _step(m, h, key_f, prim_f, dec_f, idx_f, bw_f, gw_f, cc_f,
                     N, C, D, K)
        outs.append(m.reshape(BS, N, D)[:, :C])

    output = jnp.stack(outs, axis=1)
    return output, h.reshape(BS, N, D)


# speedup vs baseline: 2.4182x; 2.1804x over previous
import functools

import jax
import jax.numpy as jnp
from jax import lax
from jax.experimental import pallas as pl
from jax.experimental.pallas import tpu as pltpu
from jax.experimental.pallas import tpu_sc as plsc

NB, BSZ, NG, BPG = 8, 4, 4, 2
L = 16
G = 4
NC, NS = 2, 16
NPS = 628
CH_MAX = NPS // G
CH_PAD = 160
BN = 80


def _sc_gather_body(N, K, D,
                    msg_f, idx_f, gout,
                    idx_all, buf, sem_g, sem_o, sem_i):
    bs = lax.axis_index("c")
    sid = lax.axis_index("s")
    n_start = sid * NPS
    nodes_here = jnp.minimum(NPS, N - n_start)
    ch_count = nodes_here // G

    row0 = (bs * NS + sid) * CH_PAD
    pltpu.async_copy(idx_f.at[pl.ds(row0, CH_PAD)], idx_all, sem_i)
    pltpu.make_async_copy(idx_f.at[pl.ds(0, CH_PAD)], idx_all, sem_i).wait()

    def obase(j):
        return (bs * N + n_start + j * G) * K

    def issue(j, p):
        @pl.when(j < ch_count)
        def _():
            pltpu.async_copy(msg_f.at[idx_all.at[j]], buf.at[p], sem_g.at[p])

    def wait_in(j, p):
        pltpu.make_async_copy(msg_f.at[idx_all.at[j]], buf.at[p],
                              sem_g.at[p]).wait()

    def wait_out(j, p):
        pltpu.make_async_copy(buf.at[p], gout.at[pl.ds(obase(j), G * K)],
                              sem_o.at[p]).wait()

    issue(0, 0)

    def chunk_iter(j, _):
        p = j % 2
        pn = (j + 1) % 2

        @pl.when((j >= 1) & (j - 1 < ch_count))
        def _():
            wait_out(j - 1, pn)

        issue(j + 1, pn)

        @pl.when(j < ch_count)
        def _():
            wait_in(j, p)
            pltpu.async_copy(buf.at[p], gout.at[pl.ds(obase(j), G * K)],
                             sem_o.at[p])

        return 0

    lax.fori_loop(0, CH_MAX, chunk_iter, 0)

    @pl.when(ch_count == CH_MAX)
    def _():
        wait_out(CH_MAX - 1, (CH_MAX - 1) % 2)


@functools.partial(jax.jit, static_argnums=(2, 3, 4))
def _sc_gather(msg_f, idx_f, N, K, D):
    mesh = plsc.VectorSubcoreMesh(core_axis_name="c", subcore_axis_name="s",
                                  num_cores=NC, num_subcores=NS)
    body = functools.partial(_sc_gather_body, N, K, D)
    BSN = msg_f.shape[0]
    return pl.kernel(
        body,
        out_type=jax.ShapeDtypeStruct((BSN * K, D), jnp.float32),
        mesh=mesh,
        compiler_params=pltpu.CompilerParams(needs_layout_passes=False),
        scratch_types=[
            pltpu.VMEM((CH_PAD, G * K), jnp.int32),
            pltpu.VMEM((2, G * K, D), jnp.float32),
            pltpu.SemaphoreType.DMA((2,)),
            pltpu.SemaphoreType.DMA((2,)),
            pltpu.SemaphoreType.DMA,
        ],
    )(msg_f, idx_f)


def _tc_math_body(NBLK, K, D, cc_ref, gout_ref, h_ref, key_ref, prim_ref,
                  dec_ref, bw_ref, gw_ref, hn_ref, mn_ref):
    i = pl.program_id(0)
    msgs = gout_ref[...].reshape(BN, K, D)
    key = key_ref[...]
    sim = jnp.sum(msgs * key[:, None, :], axis=-1)
    rt = jax.nn.sigmoid(sim)
    w = (msgs * rt[..., None]).reshape(BN, NB, BSZ, D)
    bw = bw_ref[...].reshape(BN, NB, BSZ, D)
    branch = jnp.tanh(jnp.sum(w * bw, axis=2))
    gw = gw_ref[...].reshape(BN, NG, BPG, D)
    group = jnp.tanh(jnp.sum(branch.reshape(BN, NG, BPG, D) * gw, axis=2))
    received = jnp.mean(group, axis=1)
    received = received + jnp.where((i % NBLK) == 0, cc_ref[0], 0.0)
    dec = dec_ref[...]
    hn = dec * h_ref[...] + (1.0 - dec) * received
    hn_ref[...] = hn
    mn_ref[...] = jnp.tanh(hn * prim_ref[...])


@functools.partial(jax.jit, static_argnums=(8, 9, 10))
def _tc_math(gout, h_f, key_f, prim_f, dec_f, bw_f, gw_f, cc80, N, K, D):
    NBLK = N // BN
    BSN = h_f.shape[0]
    grid = (BSN // BN,)
    body = functools.partial(_tc_math_body, NBLK, K, D)
    row = lambda i: (i, 0)
    wrow = lambda i: (i % NBLK, 0)
    return pl.pallas_call(
        body,
        grid=grid,
        in_specs=[
            pl.BlockSpec((1, BN, D), lambda i: (i // NBLK, 0, 0)),
            pl.BlockSpec((BN * K, D), row),
            pl.BlockSpec((BN, D), row),
            pl.BlockSpec((BN, D), row),
            pl.BlockSpec((BN, D), row),
            pl.BlockSpec((BN, D), row),
            pl.BlockSpec((BN * NB * BSZ, D), wrow),
            pl.BlockSpec((BN * NG * BPG, D), wrow),
        ],
        out_specs=[
            pl.BlockSpec((BN, D), row),
            pl.BlockSpec((BN, D), row),
        ],
        out_shape=[
            jax.ShapeDtypeStruct((BSN, D), jnp.float32),
            jax.ShapeDtypeStruct((BSN, D), jnp.float32),
        ],
    )(cc80, gout, h_f, key_f, prim_f, dec_f, bw_f, gw_f)


def kernel(cc_signals, h_prev, prev_messages, eff_prim, eff_key, eff_decay,
           conn_indices, branch_w, group_w):
    BS, T, C, D = cc_signals.shape
    N, K = conn_indices.shape
    n_pad = NS * NPS

    conn = conn_indices.astype(jnp.int32)
    conn = jnp.pad(conn, ((0, n_pad - N), (0, 0)))
    idx_f = (conn[None] + (jnp.arange(BS, dtype=jnp.int32) * N)[:, None, None])
    idx_f = idx_f.reshape(BS, NS, CH_MAX, G * K)
    idx_f = jnp.pad(idx_f, ((0, 0), (0, 0), (0, CH_PAD - CH_MAX), (0, 0)))
    idx_f = idx_f.reshape(BS * NS * CH_PAD, G * K)

    dec_f = jnp.broadcast_to(eff_decay[..., None], (BS, N, D)).reshape(BS * N, D)
    h_f = h_prev.reshape(BS * N, D)
    msg_f = prev_messages.reshape(BS * N, D)
    key_f = eff_key.reshape(BS * N, D)
    prim_f = eff_prim.reshape(BS * N, D)
    bw_f = branch_w.reshape(N * NB * BSZ, D)
    gw_f = group_w.reshape(N * NG * BPG, D)

    outs = []
    h, m = h_f, msg_f
    for t in range(T):
        cc80 = jnp.zeros((BS, BN, D), jnp.float32).at[:, :C].set(cc_signals[:, t])
        gout = _sc_gather(m, idx_f, N, K, D)
        h, m = _tc_math(gout, h, key_f, prim_f, dec_f, bw_f, gw_f, cc80,
                        N, K, D)
        outs.append(m.reshape(BS, N, D)[:, :C])

    output = jnp.stack(outs, axis=1)
    return output, h.reshape(BS, N, D)
